# baseline (speedup 1.0000x reference)
import functools

import jax
import jax.numpy as jnp
from jax.experimental import pallas as pl
from jax.experimental.pallas import tpu as pltpu

_CONV_DIMS = [(64, 128), (128, 256), (256, 512)]

_BLKA = 512
_BLKB = 64


def _leaky(x):
    return jnp.where(x >= 0, x, 0.01 * x)


def _mlp_body(trees_ref, *refs):
    w_refs = refs[:8]
    b_refs = refs[8:16]
    out_ref = refs[16]
    x = trees_ref[...]
    for w, b in zip(w_refs, b_refs):
        x = _leaky(jnp.dot(x.astype(jnp.bfloat16), w[...],
                           preferred_element_type=jnp.float32) + b[...])
    out_ref[...] = x


def _conv_body(x_ref, i0_ref, i1_ref, i2_ref, *refs, nblk):
    w_refs = refs[:3]
    b_refs = refs[3:6]
    out_ref = refs[6]
    x3 = x_ref[...].reshape(nblk, 64, 64)
    idx_k = [i0_ref[...], i1_ref[...], i2_ref[...]]
    node0 = jax.lax.broadcasted_iota(jnp.int32, (nblk, 64, 1), 1) == 0
    for i, (cin, cout) in enumerate(_CONV_DIMS):
        parts = []
        for k in range(3):
            g = jnp.take_along_axis(
                x3,
                jnp.broadcast_to(idx_k[k][:, None, :], (nblk, cin, 64)),
                axis=2)
            gt = jnp.swapaxes(g.astype(jnp.bfloat16), 1, 2)
            parts.append(gt.reshape(nblk * 64, cin))
        gcat = jnp.concatenate(parts, axis=1)
        h = jnp.dot(gcat, w_refs[i][...],
                    preferred_element_type=jnp.float32) + b_refs[i][...]
        z = jnp.where(node0, 0.0, h.reshape(nblk, 64, cout))
        t1 = jnp.sum(z, axis=(1, 2), keepdims=True)[:, :, 0]
        t2 = jnp.sum(z * z, axis=(1, 2), keepdims=True)[:, :, 0]
        n = 64.0 * cout
        mean = t1 / n
        var = (t2 - t1 * t1 / n) / (n - 1.0)
        rden = 1.0 / (jnp.sqrt(var) + 1e-5)
        xn = _leaky((z - mean[:, :, None]) * rden[:, :, None])
        if i < 2:
            x3 = jnp.swapaxes(xn, 1, 2)
        else:
            out_ref[...] = jnp.swapaxes(
                xn.astype(jnp.bfloat16), 1, 2).astype(jnp.float32)


@jax.jit
def kernel(trees, indexes, lw0, lb0, lw1, lb1, lw2, lb2, lw3, lb3, lw4, lb4,
           lw5, lb5, lw6, lb6, lw7, lb7, cw0, cb0, cw1, cb1, cw2, cb2):
    B = trees.shape[0]
    lws = [w.astype(jnp.bfloat16)
           for w in (lw0, lw1, lw2, lw3, lw4, lw5, lw6, lw7)]
    lbs = [b.reshape(1, -1)
           for b in (lb0, lb1, lb2, lb3, lb4, lb5, lb6, lb7)]

    grid_a = B // _BLKA
    y = pl.pallas_call(
        _mlp_body,
        grid=(grid_a,),
        in_specs=[pl.BlockSpec((_BLKA, 16), lambda i: (i, 0))]
        + [pl.BlockSpec(w.shape, lambda i: (0, 0)) for w in lws]
        + [pl.BlockSpec(b.shape, lambda i: (0, 0)) for b in lbs],
        out_specs=pl.BlockSpec((_BLKA, 4096), lambda i: (i, 0)),
        out_shape=jax.ShapeDtypeStruct((B, 4096), jnp.float32),
        compiler_params=pltpu.CompilerParams(
            dimension_semantics=("arbitrary",)),
    )(trees, *lws, *lbs)
    y2 = y.reshape(B * 64, 64)

    idx3 = indexes.reshape(B, 63, 3)
    zcol = jnp.zeros((B, 1), jnp.int32)
    idx_ks = [jnp.concatenate([zcol, idx3[:, :, k]], axis=1)
              for k in range(3)]
    cws = [cw.transpose(2, 1, 0).reshape(3 * ci, co).astype(jnp.bfloat16)
           for (ci, co), cw in zip(_CONV_DIMS, (cw0, cw1, cw2))]
    cbs = [cb.reshape(1, -1) for cb in (cb0, cb1, cb2)]

    grid_b = B // _BLKB
    out = pl.pallas_call(
        functools.partial(_conv_body, nblk=_BLKB),
        grid=(grid_b,),
        in_specs=[pl.BlockSpec((_BLKB * 64, 64), lambda i: (i, 0))]
        + [pl.BlockSpec((_BLKB, 64), lambda i: (i, 0)) for _ in range(3)]
        + [pl.BlockSpec(w.shape, lambda i: (0, 0)) for w in cws]
        + [pl.BlockSpec(b.shape, lambda i: (0, 0)) for b in cbs],
        out_specs=pl.BlockSpec((_BLKB, 512, 64), lambda i: (i, 0, 0)),
        out_shape=jax.ShapeDtypeStruct((B, 512, 64), jnp.float32),
        compiler_params=pltpu.CompilerParams(
            dimension_semantics=("arbitrary",)),
    )(y2, *idx_ks, *cws, *cbs)

    return (out, indexes)


# speedup vs baseline: 1.0401x; 1.0401x over previous
import functools

import jax
import jax.numpy as jnp
from jax.experimental import pallas as pl
from jax.experimental.pallas import tpu as pltpu

_CONV_DIMS = [(64, 128), (128, 256), (256, 512)]

_BLKA = 512
_BLKB = 64


def _leaky(x):
    return jnp.where(x >= 0, x, 0.01 * x)


def _mlp_body(trees_ref, *refs):
    w_refs = refs[:8]
    b_refs = refs[8:16]
    out_ref = refs[16]
    x = trees_ref[...]
    for w, b in zip(w_refs, b_refs):
        x = _leaky(jnp.dot(x.astype(jnp.bfloat16), w[...],
                           preferred_element_type=jnp.float32) + b[...])
    out_ref[...] = x


def _conv_body(x_ref, i0_ref, i1_ref, i2_ref, *refs, nblk):
    w_refs = refs[:3]
    b_refs = refs[3:6]
    out_ref = refs[6]
    x3 = x_ref[...].reshape(nblk, 64, 64)
    idx_k = [i0_ref[...], i1_ref[...], i2_ref[...]]
    node0 = jax.lax.broadcasted_iota(jnp.int32, (nblk, 64, 1), 1) == 0
    for i, (cin, cout) in enumerate(_CONV_DIMS):
        parts = []
        for k in range(3):
            g = jnp.take_along_axis(
                x3,
                jnp.broadcast_to(idx_k[k][:, None, :], (nblk, cin, 64)),
                axis=2)
            gt = jnp.swapaxes(g.astype(jnp.bfloat16), 1, 2)
            parts.append(gt.reshape(nblk * 64, cin))
        gcat = jnp.concatenate(parts, axis=1)
        h = jnp.dot(gcat, w_refs[i][...],
                    preferred_element_type=jnp.float32) + b_refs[i][...]
        z = jnp.where(node0, 0.0, h.reshape(nblk, 64, cout))
        t1 = jnp.sum(z, axis=(1, 2), keepdims=True)[:, :, 0]
        t2 = jnp.sum(z * z, axis=(1, 2), keepdims=True)[:, :, 0]
        n = 64.0 * cout
        mean = t1 / n
        var = (t2 - t1 * t1 / n) / (n - 1.0)
        rden = 1.0 / (jnp.sqrt(var) + 1e-5)
        xn = _leaky((z - mean[:, :, None]) * rden[:, :, None])
        if i < 2:
            x3 = jnp.swapaxes(xn, 1, 2)
        else:
            out_ref[...] = jnp.swapaxes(
                xn.astype(jnp.bfloat16), 1, 2).astype(jnp.float32)


@jax.jit
def kernel(trees, indexes, lw0, lb0, lw1, lb1, lw2, lb2, lw3, lb3, lw4, lb4,
           lw5, lb5, lw6, lb6, lw7, lb7, cw0, cb0, cw1, cb1, cw2, cb2):
    B = trees.shape[0]
    lws = [w.astype(jnp.bfloat16)
           for w in (lw0, lw1, lw2, lw3, lw4, lw5, lw6, lw7)]
    lbs = [b.reshape(1, -1)
           for b in (lb0, lb1, lb2, lb3, lb4, lb5, lb6, lb7)]

    grid_a = B // _BLKA
    y = pl.pallas_call(
        _mlp_body,
        grid=(grid_a,),
        in_specs=[pl.BlockSpec((_BLKA, 16), lambda i: (i, 0))]
        + [pl.BlockSpec(w.shape, lambda i: (0, 0)) for w in lws]
        + [pl.BlockSpec(b.shape, lambda i: (0, 0)) for b in lbs],
        out_specs=pl.BlockSpec((_BLKA, 4096), lambda i: (i, 0)),
        out_shape=jax.ShapeDtypeStruct((B, 4096), jnp.float32),
        compiler_params=pltpu.CompilerParams(
            dimension_semantics=("arbitrary",)),
    )(trees, *lws, *lbs)

    idx3 = indexes.reshape(B, 63, 3)
    zcol = jnp.zeros((B, 1), jnp.int32)
    idx_ks = [jnp.concatenate([zcol, idx3[:, :, k]], axis=1)
              for k in range(3)]
    cws = [cw.transpose(2, 1, 0).reshape(3 * ci, co).astype(jnp.bfloat16)
           for (ci, co), cw in zip(_CONV_DIMS, (cw0, cw1, cw2))]
    cbs = [cb.reshape(1, -1) for cb in (cb0, cb1, cb2)]

    grid_b = B // _BLKB
    out = pl.pallas_call(
        functools.partial(_conv_body, nblk=_BLKB),
        grid=(grid_b,),
        in_specs=[pl.BlockSpec((_BLKB, 4096), lambda i: (i, 0))]
        + [pl.BlockSpec((_BLKB, 64), lambda i: (i, 0)) for _ in range(3)]
        + [pl.BlockSpec(w.shape, lambda i: (0, 0)) for w in cws]
        + [pl.BlockSpec(b.shape, lambda i: (0, 0)) for b in cbs],
        out_specs=pl.BlockSpec((_BLKB, 512, 64), lambda i: (i, 0, 0)),
        out_shape=jax.ShapeDtypeStruct((B, 512, 64), jnp.float32),
        compiler_params=pltpu.CompilerParams(
            dimension_semantics=("arbitrary",)),
    )(y, *idx_ks, *cws, *cbs)

    return (out, indexes)


# speedup vs baseline: 1.0408x; 1.0007x over previous
import functools

import jax
import jax.numpy as jnp
from jax.experimental import pallas as pl
from jax.experimental.pallas import tpu as pltpu

_CONV_DIMS = [(64, 128), (128, 256), (256, 512)]

_BLKA = 512
_BLKB = 64


def _leaky(x):
    return jnp.where(x >= 0, x, 0.01 * x)


def _mlp_body(trees_ref, *refs):
    w_refs = refs[:8]
    b_refs = refs[8:16]
    out_ref = refs[16]
    x = trees_ref[...]
    for w, b in zip(w_refs, b_refs):
        x = _leaky(jnp.dot(x.astype(jnp.bfloat16), w[...],
                           preferred_element_type=jnp.float32) + b[...])
    out_ref[...] = x


def _conv_body(x_ref, i0_ref, i1_ref, i2_ref, *refs, nblk):
    w_refs = refs[:3]
    b_refs = refs[3:6]
    out_ref = refs[6]
    x3 = x_ref[...].reshape(nblk, 64, 64)
    idx_k = [i0_ref[...], i1_ref[...], i2_ref[...]]
    node0 = jax.lax.broadcasted_iota(jnp.int32, (nblk, 64, 1), 1) == 0
    for i, (cin, cout) in enumerate(_CONV_DIMS):
        parts = []
        for k in range(3):
            g = jnp.take_along_axis(
                x3,
                jnp.broadcast_to(idx_k[k][:, None, :], (nblk, cin, 64)),
                axis=2)
            gt = jnp.swapaxes(g.astype(jnp.bfloat16), 1, 2)
            parts.append(gt.reshape(nblk * 64, cin))
        gcat = jnp.concatenate(parts, axis=1)
        h = jnp.dot(gcat, w_refs[i][...],
                    preferred_element_type=jnp.float32) + b_refs[i][...]
        z = jnp.where(node0, 0.0, h.reshape(nblk, 64, cout))
        t1 = jnp.sum(z, axis=(1, 2), keepdims=True)[:, :, 0]
        t2 = jnp.sum(z * z, axis=(1, 2), keepdims=True)[:, :, 0]
        n = 64.0 * cout
        mean = t1 / n
        var = (t2 - t1 * t1 / n) / (n - 1.0)
        rden = 1.0 / (jnp.sqrt(var) + 1e-5)
        xn = _leaky((z - mean[:, :, None]) * rden[:, :, None])
        if i < 2:
            x3 = jnp.swapaxes(xn, 1, 2)
        else:
            out_ref[...] = jnp.swapaxes(
                xn.astype(jnp.bfloat16), 1, 2).astype(jnp.float32)


@jax.jit
def kernel(trees, indexes, lw0, lb0, lw1, lb1, lw2, lb2, lw3, lb3, lw4, lb4,
           lw5, lb5, lw6, lb6, lw7, lb7, cw0, cb0, cw1, cb1, cw2, cb2):
    B = trees.shape[0]
    lws = [w.astype(jnp.bfloat16)
           for w in (lw0, lw1, lw2, lw3, lw4, lw5, lw6, lw7)]
    lbs = [b.reshape(1, -1)
           for b in (lb0, lb1, lb2, lb3, lb4, lb5, lb6, lb7)]

    grid_a = B // _BLKA
    y = pl.pallas_call(
        _mlp_body,
        grid=(grid_a,),
        in_specs=[pl.BlockSpec((_BLKA, 16), lambda i: (i, 0))]
        + [pl.BlockSpec(w.shape, lambda i: (0, 0)) for w in lws]
        + [pl.BlockSpec(b.shape, lambda i: (0, 0)) for b in lbs],
        out_specs=pl.BlockSpec((_BLKA, 4096), lambda i: (i, 0)),
        out_shape=jax.ShapeDtypeStruct((B, 4096), jnp.float32),
        compiler_params=pltpu.CompilerParams(
            dimension_semantics=("parallel",)),
    )(trees, *lws, *lbs)

    idx3 = indexes.reshape(B, 63, 3)
    zcol = jnp.zeros((B, 1), jnp.int32)
    idx_ks = [jnp.concatenate([zcol, idx3[:, :, k]], axis=1)
              for k in range(3)]
    cws = [cw.transpose(2, 1, 0).reshape(3 * ci, co).astype(jnp.bfloat16)
           for (ci, co), cw in zip(_CONV_DIMS, (cw0, cw1, cw2))]
    cbs = [cb.reshape(1, -1) for cb in (cb0, cb1, cb2)]

    grid_b = B // _BLKB
    out = pl.pallas_call(
        functools.partial(_conv_body, nblk=_BLKB),
        grid=(grid_b,),
        in_specs=[pl.BlockSpec((_BLKB, 4096), lambda i: (i, 0))]
        + [pl.BlockSpec((_BLKB, 64), lambda i: (i, 0)) for _ in range(3)]
        + [pl.BlockSpec(w.shape, lambda i: (0, 0)) for w in cws]
        + [pl.BlockSpec(b.shape, lambda i: (0, 0)) for b in cbs],
        out_specs=pl.BlockSpec((_BLKB, 512, 64), lambda i: (i, 0, 0)),
        out_shape=jax.ShapeDtypeStruct((B, 512, 64), jnp.float32),
        compiler_params=pltpu.CompilerParams(
            dimension_semantics=("parallel",)),
    )(y, *idx_ks, *cws, *cbs)

    return (out, indexes)
